# Initial kernel scaffold; baseline (speedup 1.0000x reference)
import jax
import jax.numpy as jnp
from jax.experimental import pallas as pl


def kernel(posit_embedding, seq_length):
    raise NotImplementedError("write your pallas kernel here")


# speedup vs baseline: 1.5100x; 1.5100x over previous
import functools

import jax
import jax.numpy as jnp
from jax import lax
from jax.experimental import pallas as pl
from jax.experimental.pallas import tpu as pltpu
from jax.experimental.pallas import tpu_sc as plsc

_V = 8192
_D = 1024
_NW = 32
_RPW = _V // _NW
_C = 32
_NCH = _RPW // _C
_L = 16

_mesh = plsc.VectorSubcoreMesh(core_axis_name="c", subcore_axis_name="s")


@functools.partial(
    pl.kernel,
    out_type=jax.ShapeDtypeStruct((_V, _D), jnp.float32),
    mesh=_mesh,
    scratch_types=[
        pltpu.VMEM((_RPW,), jnp.int32),
        pltpu.VMEM((_L,), jnp.int32),
        pltpu.VMEM((_C, _D), jnp.float32),
        pltpu.VMEM((_C, _D), jnp.float32),
        pltpu.SemaphoreType.DMA,
        pltpu.SemaphoreType.DMA,
        pltpu.SemaphoreType.DMA,
        pltpu.SemaphoreType.DMA,
    ],
)
def _sc_embed(table, limit_hbm, out, idx_v, lim_v, buf0, buf1, g0, g1, s0, s1):
    wid = lax.axis_index("s") * 2 + lax.axis_index("c")
    base = wid * _RPW

    pltpu.sync_copy(limit_hbm, lim_v)
    limit = lim_v[...]
    ramp = lax.iota(jnp.int32, _L)
    for j in range(_RPW // _L):
        idx_v[pl.ds(j * _L, _L)] = jnp.minimum(ramp + (base + j * _L), limit)

    bufs = (buf0, buf1)
    gsem = (g0, g1)
    ssem = (s0, s1)
    gather_cp = [None, None]
    store_cp = [None, None]

    gather_cp[0] = pltpu.async_copy(
        table.at[idx_v.at[pl.ds(0, _C)]], bufs[0], gsem[0])
    for c in range(_NCH):
        cur = c & 1
        nxt = 1 - cur
        if c + 1 < _NCH:
            if store_cp[nxt] is not None:
                store_cp[nxt].wait()
                store_cp[nxt] = None
            gather_cp[nxt] = pltpu.async_copy(
                table.at[idx_v.at[pl.ds((c + 1) * _C, _C)]], bufs[nxt],
                gsem[nxt])
        gather_cp[cur].wait()
        store_cp[cur] = pltpu.async_copy(
            bufs[cur], out.at[pl.ds(base + c * _C, _C)], ssem[cur])
    for b in range(2):
        if store_cp[b] is not None:
            store_cp[b].wait()


def kernel(posit_embedding, seq_length):
    s = jnp.asarray(seq_length, jnp.int32)
    limit = jnp.clip(s - 1, 0, _V - 1)
    limit_vec = jnp.broadcast_to(limit, (_L,)).astype(jnp.int32)
    out = _sc_embed(posit_embedding, limit_vec)
    return out[None, :, :]
